# baseline (speedup 1.0000x reference)
import functools

import jax
import jax.numpy as jnp
from jax import lax
from jax.experimental import pallas as pl
from jax.experimental.pallas import tpu as pltpu
from jax.experimental.pallas import tpu_sc as plsc

VOCAB = 1000000
EMBED = 64
CTX = 200
BATCH = 1024

NUM_WORKERS = 32
IDX_TOTAL = BATCH * CTX
PER_WORKER = IDX_TOTAL // NUM_WORKERS
CHUNK = 100
CHUNKS = PER_WORKER // CHUNK
VREGS_PER_ROW = EMBED // 16
NBUF = 4
LEAD = NBUF - 1
OUTER = CHUNKS // NBUF
ROWS_STEP = 10


def _emb_body(x_hbm, pos_hbm, table_hbm, out_hbm,
              idx_v, pos_v, rows_v, *sems):
    gsems = sems[:NBUF]
    wsems = sems[NBUF:]
    cidx = lax.axis_index("c")
    sidx = lax.axis_index("s")
    wid = sidx * 2 + cidx

    pltpu.sync_copy(x_hbm.at[wid], idx_v)
    pltpu.sync_copy(pos_hbm, pos_v)

    def gather_start(ch, b):
        pltpu.async_copy(table_hbm.at[idx_v.at[ch]], rows_v.at[b], gsems[b])

    def gather_wait(ch, b):
        pltpu.make_async_copy(
            table_hbm.at[idx_v.at[ch]], rows_v.at[b], gsems[b]).wait()

    def wb_start(ch, b):
        pltpu.async_copy(rows_v.at[b], out_hbm.at[wid * CHUNKS + ch],
                         wsems[b])

    def wb_wait(ch, b):
        pltpu.make_async_copy(
            rows_v.at[b], out_hbm.at[wid * CHUNKS + ch], wsems[b]).wait()

    for b in range(LEAD):
        gather_start(b, b)

    def outer_body(o, _):
        for b in range(NBUF):
            ch = o * NBUF + b
            bg = (b + LEAD) % NBUF
            if b == 0:
                pl.when(o > 0)(lambda: wb_wait(ch - 1, bg))
                gather_start(ch + LEAD, bg)
            else:
                wb_wait(ch - 1, bg)
                pl.when(o < OUTER - 1)(lambda: gather_start(ch + LEAD, bg))

            gather_wait(ch, b)

            pos_base = (b % 2) * CHUNK

            def add_rows(k, _, b=b, pos_base=pos_base):
                r0 = k * ROWS_STEP
                for rs in range(ROWS_STEP):
                    for j in range(VREGS_PER_ROW):
                        sl = pl.ds(j * 16, 16)
                        rows_v[b, r0 + rs, sl] = (
                            rows_v[b, r0 + rs, sl]
                            + pos_v[pos_base + r0 + rs, sl])
                return 0

            lax.fori_loop(0, CHUNK // ROWS_STEP, add_rows, 0)

            wb_start(ch, b)
        return 0

    lax.fori_loop(0, OUTER, outer_body, 0)
    wb_wait(CHUNKS - 1, (CHUNKS - 1) % NBUF)


@jax.jit
def _emb_call(x_grp, table, pos_enc):
    mesh = plsc.VectorSubcoreMesh(core_axis_name="c", subcore_axis_name="s")
    kern = functools.partial(
        pl.kernel,
        mesh=mesh,
        out_type=jax.ShapeDtypeStruct((IDX_TOTAL // CHUNK, CHUNK, EMBED),
                                      jnp.float32),
        scratch_types=[
            pltpu.VMEM((CHUNKS, CHUNK), jnp.int32),
            pltpu.VMEM((CTX, EMBED), jnp.float32),
            pltpu.VMEM((NBUF, CHUNK, EMBED), jnp.float32),
        ] + [pltpu.SemaphoreType.DMA] * (2 * NBUF),
        compiler_params=pltpu.CompilerParams(use_tc_tiling_on_sc=False),
    )(_emb_body)
    return kern(x_grp, pos_enc, table)


def kernel(x, table, pos_enc):
    x_grp = x.reshape(NUM_WORKERS, CHUNKS, CHUNK).astype(jnp.int32)
    out = _emb_call(x_grp, table, pos_enc)
    return out.reshape(BATCH, CTX, EMBED)
